# baseline (speedup 1.0000x reference)
import functools

import jax
import jax.numpy as jnp
from jax import lax
from jax.experimental import pallas as pl
from jax.experimental.pallas import tpu as pltpu
from jax.experimental.pallas import tpu_sc as plsc

_NUM_FIELDS = 26
_FIELD_DIM = 38462
_EMBED_DIM = 16
_BATCH = 4096
_TOTAL = _BATCH * _NUM_FIELDS
_NUM_WORKERS = 32
_PER_W = _TOTAL // _NUM_WORKERS
_CHUNK = 128
_NCHUNK = _PER_W // _CHUNK
_LANES = 16


def _body(idx_hbm, offt_hbm, table_hbm, out_hbm, idx_v, offt_v, rows_v, sem):
    wid = lax.axis_index("s") * 2 + lax.axis_index("c")
    base = wid * _PER_W

    pltpu.sync_copy(idx_hbm.at[pl.ds(base, _PER_W)], idx_v)
    pltpu.sync_copy(offt_hbm, offt_v)

    def add_vec(i, carry):
        sl = pl.ds(i * _LANES, _LANES)
        idx_v[sl] = idx_v[sl] + offt_v[sl]
        return carry

    lax.fori_loop(0, _PER_W // _LANES, add_vec, 0)

    copies = []
    for c in range(_NCHUNK):
        copies.append(
            pltpu.async_copy(
                table_hbm.at[idx_v.at[pl.ds(c * _CHUNK, _CHUNK)]],
                rows_v.at[pl.ds(c * _CHUNK, _CHUNK)],
                sem,
            )
        )
    for cp in copies:
        cp.wait()

    pltpu.sync_copy(rows_v, out_hbm.at[pl.ds(wid * _PER_W, _PER_W)])


@jax.jit
def _fmembedding(idx_flat, table, offt):
    mesh = plsc.VectorSubcoreMesh(
        core_axis_name="c", subcore_axis_name="s", num_cores=2, num_subcores=16
    )
    run = functools.partial(
        pl.kernel,
        out_type=jax.ShapeDtypeStruct((_TOTAL, _EMBED_DIM), jnp.float32),
        mesh=mesh,
        scratch_types=[
            pltpu.VMEM((_PER_W,), jnp.int32),
            pltpu.VMEM((_PER_W,), jnp.int32),
            pltpu.VMEM((_PER_W, _EMBED_DIM), jnp.float32),
            pltpu.SemaphoreType.DMA,
        ],
        compiler_params=pltpu.CompilerParams(use_tc_tiling_on_sc=False),
    )(_body)
    return run(idx_flat, offt, table)


def kernel(input_x, table, offsets):
    idx_flat = input_x.reshape(_TOTAL)
    offt = jnp.tile(offsets, _PER_W // _NUM_FIELDS)
    out = _fmembedding(idx_flat, table, offt)
    return out.reshape(_BATCH, _NUM_FIELDS, _EMBED_DIM)
